# Initial kernel scaffold; baseline (speedup 1.0000x reference)
import jax
import jax.numpy as jnp
from jax.experimental import pallas as pl


def kernel(x, moe_We, moe_be, moe_Wg, moe_bg, Wqkv, bqkv, Wo, bo, ln1_g, ln1_b, W1, b1, W2, b2, ln2_g, ln2_b, Wf, bf, Wc, bc, Wk1, bk1, Wk2, bk2, Wout, bout):
    raise NotImplementedError("write your pallas kernel here")


# speedup vs baseline: 1.5140x; 1.5140x over previous
import functools

import jax
import jax.numpy as jnp
from jax import lax
from jax.experimental import pallas as pl
from jax.experimental.pallas import tpu as pltpu

NHEAD = 4
H = 1024
DH = H // NHEAD
BLK = 512


def _dot_t(a, w):
    return lax.dot_general(a, w, (((1,), (1,)), ((), ())),
                           preferred_element_type=jnp.float32)


def _dot(a, w):
    return jnp.dot(a, w, preferred_element_type=jnp.float32)


def _gate_body(xrow_ref, wg_ref, bg_ref, idx_ref):
    logits = _dot_t(xrow_ref[...], wg_ref[...]) + bg_ref[...]
    iota = lax.broadcasted_iota(jnp.int32, logits.shape, 1)
    m1 = jnp.max(logits)
    i1 = jnp.min(jnp.where(logits >= m1, iota, 8))
    masked = jnp.where(iota == i1, -jnp.inf, logits)
    m2 = jnp.max(masked)
    i2 = jnp.min(jnp.where(masked >= m2, iota, 8))
    idx_ref[0] = i1.astype(jnp.int32)
    idx_ref[1] = i2.astype(jnp.int32)


def _gate(xrow, wg, bg):
    return pl.pallas_call(
        _gate_body,
        out_shape=jax.ShapeDtypeStruct((2,), jnp.int32),
        out_specs=pl.BlockSpec(memory_space=pltpu.SMEM),
    )(xrow, wg, bg)


def _avg_body(idx_ref, we_ref, be_ref, xrow_ref, wavg_ref, bavg_ref,
              xnext_ref):
    k = pl.program_id(0)

    @pl.when(k == 0)
    def _():
        wavg_ref[...] = jnp.zeros_like(wavg_ref)
        bavg_ref[...] = jnp.zeros_like(bavg_ref)
        xnext_ref[...] = jnp.zeros_like(xnext_ref)

    w = we_ref[0]
    b = be_ref[0]
    wavg_ref[...] += 0.5 * w
    bavg_ref[...] += 0.5 * b
    xnext_ref[...] += 0.5 * (_dot_t(xrow_ref[...], w) + b)


def _avg(idx, we, be, xrow):
    grid_spec = pltpu.PrefetchScalarGridSpec(
        num_scalar_prefetch=1,
        grid=(2,),
        in_specs=[
            pl.BlockSpec((1, H, H), lambda k, idx_ref: (idx_ref[k], 0, 0)),
            pl.BlockSpec((1, 1, H), lambda k, idx_ref: (idx_ref[k], 0, 0)),
            pl.BlockSpec((1, H), lambda k, idx_ref: (0, 0)),
        ],
        out_specs=[
            pl.BlockSpec((H, H), lambda k, idx_ref: (0, 0)),
            pl.BlockSpec((1, H), lambda k, idx_ref: (0, 0)),
            pl.BlockSpec((1, H), lambda k, idx_ref: (0, 0)),
        ],
    )
    return pl.pallas_call(
        _avg_body,
        grid_spec=grid_spec,
        out_shape=[
            jax.ShapeDtypeStruct((H, H), jnp.float32),
            jax.ShapeDtypeStruct((1, H), jnp.float32),
            jax.ShapeDtypeStruct((1, H), jnp.float32),
        ],
    )(idx, we, be.reshape(8, 1, H), xrow)


def _moe_qkv_body(x_ref, w0_ref, b0_ref, w1_ref, b1_ref, w2_ref, b2_ref,
                  wqkv_ref, bqkv_ref, xmoe_ref, qkv_ref):
    y = x_ref[...]
    y = _dot_t(y, w0_ref[...]) + b0_ref[...]
    y = _dot_t(y, w1_ref[...]) + b1_ref[...]
    y = _dot_t(y, w2_ref[...]) + b2_ref[...]
    xmoe_ref[...] = y
    qkv_ref[...] = _dot_t(y, wqkv_ref[...]) + bqkv_ref[...]


def _moe_qkv(x, w0, b0, w1, b1, w2, b2, wqkv, bqkv):
    S = x.shape[0]
    full = lambda t: (0, 0)
    return pl.pallas_call(
        _moe_qkv_body,
        grid=(S // BLK,),
        in_specs=[
            pl.BlockSpec((BLK, H), lambda t: (t, 0)),
            pl.BlockSpec((H, H), full), pl.BlockSpec((1, H), full),
            pl.BlockSpec((H, H), full), pl.BlockSpec((1, H), full),
            pl.BlockSpec((H, H), full), pl.BlockSpec((1, H), full),
            pl.BlockSpec((3 * H, H), full), pl.BlockSpec((1, 3 * H), full),
        ],
        out_specs=[
            pl.BlockSpec((BLK, H), lambda t: (t, 0)),
            pl.BlockSpec((BLK, 3 * H), lambda t: (t, 0)),
        ],
        out_shape=[
            jax.ShapeDtypeStruct((S, H), jnp.float32),
            jax.ShapeDtypeStruct((S, 3 * H), jnp.float32),
        ],
    )(x, w0, b0, w1, b1, w2, b2, wqkv, bqkv)


def _attn_body(q_ref, k_ref, v_ref, o_ref):
    q = q_ref[...]
    k = k_ref[...]
    s = _dot_t(q, k) * (1.0 / (DH ** 0.5))
    m = jnp.max(s, axis=-1, keepdims=True)
    e = jnp.exp(s - m)
    p = e / jnp.sum(e, axis=-1, keepdims=True)
    o_ref[...] = _dot(p, v_ref[...])


def _attention(qkv, S):
    return pl.pallas_call(
        _attn_body,
        grid=(NHEAD, S // BLK),
        in_specs=[
            pl.BlockSpec((BLK, DH), lambda h, t: (t, h)),
            pl.BlockSpec((S, DH), lambda h, t: (0, NHEAD + h)),
            pl.BlockSpec((S, DH), lambda h, t: (0, 2 * NHEAD + h)),
        ],
        out_specs=pl.BlockSpec((BLK, DH), lambda h, t: (t, h)),
        out_shape=jax.ShapeDtypeStruct((S, H), jnp.float32),
    )(qkv, qkv, qkv)


def _ln(x, g, b):
    m = jnp.mean(x, axis=-1, keepdims=True)
    c = x - m
    v = jnp.mean(c * c, axis=-1, keepdims=True)
    return c * lax.rsqrt(v + 1e-5) * g + b


def _post_body(xmoe_ref, ao_ref, wo_ref, bo_ref, g1_ref, be1_ref,
               w1_ref, b1_ref, w2_ref, b2_ref, g2_ref, be2_ref, o_ref):
    t = _dot_t(ao_ref[...], wo_ref[...]) + bo_ref[...]
    x = _ln(xmoe_ref[...] + t, g1_ref[...], be1_ref[...])
    h = jnp.maximum(_dot_t(x, w1_ref[...]) + b1_ref[...], 0.0)
    f = _dot_t(h, w2_ref[...]) + b2_ref[...]
    o_ref[...] = _ln(x + f, g2_ref[...], be2_ref[...])


def _post_attn(xmoe, ao, wo, bo, g1, be1, w1, b1, w2, b2, g2, be2):
    S = xmoe.shape[0]
    full = lambda t: (0, 0)
    blk = pl.BlockSpec((BLK, H), lambda t: (t, 0))
    return pl.pallas_call(
        _post_body,
        grid=(S // BLK,),
        in_specs=[
            blk, blk,
            pl.BlockSpec((H, H), full), pl.BlockSpec((1, H), full),
            pl.BlockSpec((1, H), full), pl.BlockSpec((1, H), full),
            pl.BlockSpec((2 * H, H), full), pl.BlockSpec((1, 2 * H), full),
            pl.BlockSpec((H, 2 * H), full), pl.BlockSpec((1, H), full),
            pl.BlockSpec((1, H), full), pl.BlockSpec((1, H), full),
        ],
        out_specs=blk,
        out_shape=jax.ShapeDtypeStruct((S, H), jnp.float32),
    )(xmoe, ao, wo, bo, g1, be1, w1, b1, w2, b2, g2, be2)


def _tail_body(x_ref, wf_ref, bf_ref, wc_ref, bc_ref, wk1_ref, bk1_ref,
               wk2_ref, bk2_ref, wout_ref, bout_ref, o_ref):
    x = x_ref[...]
    x = _dot_t(x, wf_ref[...]) + bf_ref[...]
    x = _dot_t(x, wc_ref[...]) + bc_ref[...]
    h = jnp.maximum(_dot_t(x, wk1_ref[...]) + bk1_ref[...], 0.0)
    x = _dot_t(h, wk2_ref[...]) + bk2_ref[...]
    o_ref[...] = _dot_t(x, wout_ref[...]) + bout_ref[...]


def _tail(x, wf, bf, wc, bc, wk1, bk1, wk2, bk2, wout, bout):
    S = x.shape[0]
    full = lambda t: (0, 0)
    wspec = pl.BlockSpec((H, H), full)
    bspec = pl.BlockSpec((1, H), full)
    blk = pl.BlockSpec((BLK, H), lambda t: (t, 0))
    return pl.pallas_call(
        _tail_body,
        grid=(S // BLK,),
        in_specs=[blk, wspec, bspec, wspec, bspec, wspec, bspec,
                  wspec, bspec, wspec, bspec],
        out_specs=blk,
        out_shape=jax.ShapeDtypeStruct((S, H), jnp.float32),
    )(x, wf, bf, wc, bc, wk1, bk1, wk2, bk2, wout, bout)


def kernel(x, moe_We, moe_be, moe_Wg, moe_bg, Wqkv, bqkv, Wo, bo, ln1_g,
           ln1_b, W1, b1, W2, b2, ln2_g, ln2_b, Wf, bf, Wc, bc, Wk1, bk1,
           Wk2, bk2, Wout, bout):
    S = x.shape[0]
    row = lambda v: v.reshape(1, -1)

    xrow = x[0:1]
    wavg, bavg = [], []
    for i in range(3):
        idx = _gate(xrow, moe_Wg[i], row(moe_bg[i]))
        w_i, b_i, xrow = _avg(idx, moe_We[i], moe_be[i], xrow)
        wavg.append(w_i)
        bavg.append(b_i)

    xmoe, qkv = _moe_qkv(x, wavg[0], bavg[0], wavg[1], bavg[1],
                         wavg[2], bavg[2], Wqkv, row(bqkv))
    ao = _attention(qkv, S)
    xp = _post_attn(xmoe, ao, Wo, row(bo), row(ln1_g), row(ln1_b),
                    W1, row(b1), W2, row(b2), row(ln2_g), row(ln2_b))
    return _tail(xp, Wf, row(bf), Wc, row(bc), Wk1, row(bk1),
                 Wk2, row(bk2), Wout, row(bout))


# speedup vs baseline: 1.6108x; 1.0640x over previous
import functools

import jax
import jax.numpy as jnp
from jax import lax
from jax.experimental import pallas as pl
from jax.experimental.pallas import tpu as pltpu

NHEAD = 4
H = 1024
DH = H // NHEAD
BLK = 512


def _dot_t(a, w):
    return lax.dot_general(a, w, (((1,), (1,)), ((), ())),
                           preferred_element_type=jnp.float32)


def _dot(a, w):
    return jnp.dot(a, w, preferred_element_type=jnp.float32)


def _gate_body(xrow_ref, wg_ref, bg_ref, idx_ref):
    logits = _dot_t(xrow_ref[...], wg_ref[...]) + bg_ref[...]
    iota = lax.broadcasted_iota(jnp.int32, logits.shape, 1)
    m1 = jnp.max(logits)
    i1 = jnp.min(jnp.where(logits >= m1, iota, 8))
    masked = jnp.where(iota == i1, -jnp.inf, logits)
    m2 = jnp.max(masked)
    i2 = jnp.min(jnp.where(masked >= m2, iota, 8))
    idx_ref[0] = i1.astype(jnp.int32)
    idx_ref[1] = i2.astype(jnp.int32)


def _gate(xrow, wg, bg):
    return pl.pallas_call(
        _gate_body,
        out_shape=jax.ShapeDtypeStruct((2,), jnp.int32),
        out_specs=pl.BlockSpec(memory_space=pltpu.SMEM),
    )(xrow, wg, bg)


def _avg_body(idx_ref, we_ref, be_ref, xrow_ref, wavg_ref, bavg_ref,
              xnext_ref):
    k = pl.program_id(0)

    @pl.when(k == 0)
    def _():
        wavg_ref[...] = jnp.zeros_like(wavg_ref)
        bavg_ref[...] = jnp.zeros_like(bavg_ref)
        xnext_ref[...] = jnp.zeros_like(xnext_ref)

    w = we_ref[0]
    b = be_ref[0]
    wavg_ref[...] += 0.5 * w
    bavg_ref[...] += 0.5 * b
    xnext_ref[...] += 0.5 * (_dot_t(xrow_ref[...], w) + b)


def _avg(idx, we, be, xrow):
    grid_spec = pltpu.PrefetchScalarGridSpec(
        num_scalar_prefetch=1,
        grid=(2,),
        in_specs=[
            pl.BlockSpec((1, H, H), lambda k, idx_ref: (idx_ref[k], 0, 0)),
            pl.BlockSpec((1, 1, H), lambda k, idx_ref: (idx_ref[k], 0, 0)),
            pl.BlockSpec((1, H), lambda k, idx_ref: (0, 0)),
        ],
        out_specs=[
            pl.BlockSpec((H, H), lambda k, idx_ref: (0, 0)),
            pl.BlockSpec((1, H), lambda k, idx_ref: (0, 0)),
            pl.BlockSpec((1, H), lambda k, idx_ref: (0, 0)),
        ],
    )
    return pl.pallas_call(
        _avg_body,
        grid_spec=grid_spec,
        out_shape=[
            jax.ShapeDtypeStruct((H, H), jnp.float32),
            jax.ShapeDtypeStruct((1, H), jnp.float32),
            jax.ShapeDtypeStruct((1, H), jnp.float32),
        ],
    )(idx, we, be.reshape(8, 1, H), xrow)


def _compose_body(w0_ref, b0_ref, w1_ref, b1_ref, w2_ref, b2_ref,
                  wc_ref, bc_ref):
    w21 = _dot(w2_ref[...], w1_ref[...])
    wc_ref[...] = _dot(w21, w0_ref[...])
    b01 = _dot_t(b0_ref[...], w1_ref[...]) + b1_ref[...]
    bc_ref[...] = _dot_t(b01, w2_ref[...]) + b2_ref[...]


def _compose(w0, b0, w1, b1, w2, b2):
    return pl.pallas_call(
        _compose_body,
        out_shape=[
            jax.ShapeDtypeStruct((H, H), jnp.float32),
            jax.ShapeDtypeStruct((1, H), jnp.float32),
        ],
    )(w0, b0, w1, b1, w2, b2)


def _moe_qkv_body(x_ref, wc_ref, bc_ref, wqkv_ref, bqkv_ref,
                  xmoe_ref, qkv_ref):
    y = _dot_t(x_ref[...], wc_ref[...]) + bc_ref[...]
    xmoe_ref[...] = y
    qkv_ref[...] = _dot_t(y, wqkv_ref[...]) + bqkv_ref[...]


def _moe_qkv(x, wc, bc, wqkv, bqkv):
    S = x.shape[0]
    full = lambda t: (0, 0)
    return pl.pallas_call(
        _moe_qkv_body,
        grid=(S // BLK,),
        in_specs=[
            pl.BlockSpec((BLK, H), lambda t: (t, 0)),
            pl.BlockSpec((H, H), full), pl.BlockSpec((1, H), full),
            pl.BlockSpec((3 * H, H), full), pl.BlockSpec((1, 3 * H), full),
        ],
        out_specs=[
            pl.BlockSpec((BLK, H), lambda t: (t, 0)),
            pl.BlockSpec((BLK, 3 * H), lambda t: (t, 0)),
        ],
        out_shape=[
            jax.ShapeDtypeStruct((S, H), jnp.float32),
            jax.ShapeDtypeStruct((S, 3 * H), jnp.float32),
        ],
        compiler_params=pltpu.CompilerParams(
            dimension_semantics=("parallel",)),
    )(x, wc, bc, wqkv, bqkv)


def _attn_body(q_ref, k_ref, v_ref, o_ref):
    q = q_ref[...] * (1.0 / (DH ** 0.5))
    k = k_ref[...]
    s = _dot_t(q, k)
    m = jnp.max(s, axis=-1, keepdims=True)
    e = jnp.exp(s - m)
    r = 1.0 / jnp.sum(e, axis=-1, keepdims=True)
    o_ref[...] = _dot(e, v_ref[...]) * r


def _attention(qkv, S):
    return pl.pallas_call(
        _attn_body,
        grid=(NHEAD, S // BLK),
        in_specs=[
            pl.BlockSpec((BLK, DH), lambda h, t: (t, h)),
            pl.BlockSpec((S, DH), lambda h, t: (0, NHEAD + h)),
            pl.BlockSpec((S, DH), lambda h, t: (0, 2 * NHEAD + h)),
        ],
        out_specs=pl.BlockSpec((BLK, DH), lambda h, t: (t, h)),
        out_shape=jax.ShapeDtypeStruct((S, H), jnp.float32),
        compiler_params=pltpu.CompilerParams(
            dimension_semantics=("parallel", "parallel")),
    )(qkv, qkv, qkv)


def _ln(x, g, b):
    m = jnp.mean(x, axis=-1, keepdims=True)
    c = x - m
    v = jnp.mean(c * c, axis=-1, keepdims=True)
    return c * lax.rsqrt(v + 1e-5) * g + b


def _post_body(xmoe_ref, ao_ref, wo_ref, bo_ref, g1_ref, be1_ref,
               w1_ref, b1_ref, w2_ref, b2_ref, g2_ref, be2_ref, o_ref):
    t = _dot_t(ao_ref[...], wo_ref[...]) + bo_ref[...]
    x = _ln(xmoe_ref[...] + t, g1_ref[...], be1_ref[...])
    h = jnp.maximum(_dot_t(x, w1_ref[...]) + b1_ref[...], 0.0)
    f = _dot_t(h, w2_ref[...]) + b2_ref[...]
    o_ref[...] = _ln(x + f, g2_ref[...], be2_ref[...])


def _post_attn(xmoe, ao, wo, bo, g1, be1, w1, b1, w2, b2, g2, be2):
    S = xmoe.shape[0]
    full = lambda t: (0, 0)
    blk = pl.BlockSpec((BLK, H), lambda t: (t, 0))
    return pl.pallas_call(
        _post_body,
        grid=(S // BLK,),
        in_specs=[
            blk, blk,
            pl.BlockSpec((H, H), full), pl.BlockSpec((1, H), full),
            pl.BlockSpec((1, H), full), pl.BlockSpec((1, H), full),
            pl.BlockSpec((2 * H, H), full), pl.BlockSpec((1, 2 * H), full),
            pl.BlockSpec((H, 2 * H), full), pl.BlockSpec((1, H), full),
            pl.BlockSpec((1, H), full), pl.BlockSpec((1, H), full),
        ],
        out_specs=blk,
        out_shape=jax.ShapeDtypeStruct((S, H), jnp.float32),
        compiler_params=pltpu.CompilerParams(
            dimension_semantics=("parallel",)),
    )(xmoe, ao, wo, bo, g1, be1, w1, b1, w2, b2, g2, be2)


def _tail_body(x_ref, wf_ref, bf_ref, wc_ref, bc_ref, wk1_ref, bk1_ref,
               wk2_ref, bk2_ref, wout_ref, bout_ref, o_ref):
    x = x_ref[...]
    x = _dot_t(x, wf_ref[...]) + bf_ref[...]
    x = _dot_t(x, wc_ref[...]) + bc_ref[...]
    h = jnp.maximum(_dot_t(x, wk1_ref[...]) + bk1_ref[...], 0.0)
    x = _dot_t(h, wk2_ref[...]) + bk2_ref[...]
    o_ref[...] = _dot_t(x, wout_ref[...]) + bout_ref[...]


def _tail(x, wf, bf, wc, bc, wk1, bk1, wk2, bk2, wout, bout):
    S = x.shape[0]
    full = lambda t: (0, 0)
    wspec = pl.BlockSpec((H, H), full)
    bspec = pl.BlockSpec((1, H), full)
    blk = pl.BlockSpec((BLK, H), lambda t: (t, 0))
    return pl.pallas_call(
        _tail_body,
        grid=(S // BLK,),
        in_specs=[blk, wspec, bspec, wspec, bspec, wspec, bspec,
                  wspec, bspec, wspec, bspec],
        out_specs=blk,
        out_shape=jax.ShapeDtypeStruct((S, H), jnp.float32),
        compiler_params=pltpu.CompilerParams(
            dimension_semantics=("parallel",)),
    )(x, wf, bf, wc, bc, wk1, bk1, wk2, bk2, wout, bout)


def kernel(x, moe_We, moe_be, moe_Wg, moe_bg, Wqkv, bqkv, Wo, bo, ln1_g,
           ln1_b, W1, b1, W2, b2, ln2_g, ln2_b, Wf, bf, Wc, bc, Wk1, bk1,
           Wk2, bk2, Wout, bout):
    S = x.shape[0]
    row = lambda v: v.reshape(1, -1)

    xrow = x[0:1]
    wavg, bavg = [], []
    for i in range(3):
        idx = _gate(xrow, moe_Wg[i], row(moe_bg[i]))
        w_i, b_i, xrow = _avg(idx, moe_We[i], moe_be[i], xrow)
        wavg.append(w_i)
        bavg.append(b_i)

    wc_moe, bc_moe = _compose(wavg[0], bavg[0], wavg[1], bavg[1],
                              wavg[2], bavg[2])
    xmoe, qkv = _moe_qkv(x, wc_moe, bc_moe, Wqkv, row(bqkv))
    ao = _attention(qkv, S)
    xp = _post_attn(xmoe, ao, Wo, row(bo), row(ln1_g), row(ln1_b),
                    W1, row(b1), W2, row(b2), row(ln2_g), row(ln2_b))
    return _tail(xp, Wf, row(bf), Wc, row(bc), Wk1, row(bk1),
                 Wk2, row(bk2), Wout, row(bout))
